# Initial kernel scaffold; baseline (speedup 1.0000x reference)
import jax
import jax.numpy as jnp
from jax.experimental import pallas as pl


def kernel(x, edge_index, batch, Wl1, Wr1, att1, Wl2, Wr2, att2, Wl3, Wr3, att3, G1, g1b, G2, g2b, M1, ln1w, ln1b, M2, ln2w, ln2b, M3, ln3w, ln3b, M4, ln4w, ln4b, M5, M5b):
    raise NotImplementedError("write your pallas kernel here")


# speedup vs baseline: 2.1703x; 2.1703x over previous
import functools

import jax
import jax.numpy as jnp
from jax import lax
from jax.experimental import pallas as pl
from jax.experimental.pallas import tpu as pltpu
from jax.experimental.pallas import tpu_sc as plsc

N = 10000
E = 50000
G = 16
H = 4

NC, NS = 2, 16
NW = NC * NS
NPW = 320
NPAD = NW * NPW
EMAX = 2048
BB = 8
ZR = 16


def _mm_body(x_ref, w_ref, o_ref):
    o_ref[...] = jnp.dot(x_ref[...], w_ref[...],
                         preferred_element_type=jnp.float32)


def _matmul(x, w, bm=1024, bn=1024):
    m, k = x.shape
    _, n = w.shape
    return pl.pallas_call(
        _mm_body,
        grid=(m // bm, n // bn),
        in_specs=[pl.BlockSpec((bm, k), lambda i, j: (i, 0)),
                  pl.BlockSpec((k, bn), lambda i, j: (0, j))],
        out_specs=pl.BlockSpec((bm, bn), lambda i, j: (i, j)),
        out_shape=jax.ShapeDtypeStruct((m, n), jnp.float32),
    )(x, w)


def _edge_conv(xl, xr, att_flat, src_bins, dst_bins, C):
    HC = H * C
    mesh = plsc.VectorSubcoreMesh(core_axis_name="c", subcore_axis_name="s",
                                  num_cores=NC, num_subcores=NS)

    @functools.partial(
        pl.kernel,
        out_type=jax.ShapeDtypeStruct((NPAD, C), jnp.float32),
        mesh=mesh,
        compiler_params=pltpu.CompilerParams(needs_layout_passes=False),
        scratch_types=[
            pltpu.VMEM((HC,), jnp.float32),
            pltpu.VMEM((EMAX,), jnp.int32),
            pltpu.VMEM((EMAX,), jnp.int32),
            pltpu.VMEM((BB, HC), jnp.float32),
            pltpu.VMEM((BB, HC), jnp.float32),
            pltpu.VMEM((HC,), jnp.float32),
            pltpu.VMEM((C,), jnp.float32),
            pltpu.VMEM((16,), jnp.float32),
            pltpu.VMEM((ZR, C), jnp.float32),
            pltpu.SemaphoreType.DMA,
        ],
    )
    def kern(xl_hbm, xr_hbm, att_hbm, srcb_hbm, dstb_hbm, h_hbm,
             att_v, src_v, dst_v, rowsL, rowsR, accbuf, hbuf, dref, zbuf, sem):
        wid = lax.axis_index("s") * NC + lax.axis_index("c")
        lo = wid * NPW
        lane = lax.iota(jnp.int32, 16)
        zv = jnp.zeros((16,), jnp.float32)

        pltpu.sync_copy(att_hbm, att_v)
        pltpu.sync_copy(srcb_hbm.at[wid], src_v)
        pltpu.sync_copy(dstb_hbm.at[wid], dst_v)

        for r in range(ZR):
            def zrow(kk, _, r=r):
                zbuf[r, pl.ds(kk * 16, 16)] = zv
                return 0
            lax.fori_loop(0, C // 16, zrow, 0)
        def prez(t, _):
            pltpu.sync_copy(zbuf, h_hbm.at[pl.ds(lo + t * ZR, ZR)])
            return 0
        lax.fori_loop(0, NPW // ZR, prez, 0)

        def zacc(kk, _):
            accbuf[pl.ds(kk * 16, 16)] = zv
            return 0
        lax.fori_loop(0, HC // 16, zacc, 0)
        dref[...] = zv

        def flush(cur):
            dv = dref[...]
            invs = []
            for hh in range(H):
                dh = jnp.sum(jnp.where(lane == hh, dv, 0.0))
                invs.append(0.25 / (jnp.full((16,), dh) + 1e-16))
            def fin(kk, _):
                o = kk * 16
                s = (accbuf[pl.ds(o, 16)] * invs[0]
                     + accbuf[pl.ds(C + o, 16)] * invs[1]
                     + accbuf[pl.ds(2 * C + o, 16)] * invs[2]
                     + accbuf[pl.ds(3 * C + o, 16)] * invs[3])
                hbuf[pl.ds(o, 16)] = jnp.maximum(s, 0.0)
                return 0
            lax.fori_loop(0, C // 16, fin, 0)
            pltpu.sync_copy(hbuf, h_hbm.at[cur])
            def zacc2(kk, _):
                accbuf[pl.ds(kk * 16, 16)] = zv
                return 0
            lax.fori_loop(0, HC // 16, zacc2, 0)
            dref[...] = zv

        def gbody(g, cur):
            dvch = dst_v[pl.ds(g * 16, 16)]
            for half in range(16 // BB):
                base = g * 16 + half * BB
                idxL = src_v.at[pl.ds(base, BB)]
                idxR = dst_v.at[pl.ds(base, BB)]
                cpL = pltpu.async_copy(xl_hbm.at[idxL], rowsL, sem)
                cpR = pltpu.async_copy(xr_hbm.at[idxR], rowsR, sem)
                cpL.wait()
                cpR.wait()
                for j in range(BB):
                    dst_e = dvch[half * BB + j]
                    do_flush = (dst_e != cur) & (cur >= 0) & (cur < N)

                    @pl.when(do_flush)
                    def _(cur=cur):
                        flush(cur)

                    cur = dst_e

                    @pl.when(dst_e < N)
                    def _(j=j):
                        for hh in range(H):
                            off0 = hh * C
                            def abody(kk, pv, j=j, off0=off0):
                                off = off0 + kk * 16
                                lv = rowsL[j, pl.ds(off, 16)]
                                rv = rowsR[j, pl.ds(off, 16)]
                                s = lv + rv
                                ls = jnp.where(s > 0, s, 0.2 * s)
                                return pv + ls * att_v[pl.ds(off, 16)]
                            pv = lax.fori_loop(0, C // 16, abody, zv)
                            expav = jnp.exp(jnp.full((16,), jnp.sum(pv)))
                            dref[...] = dref[...] + jnp.where(lane == hh,
                                                              expav, 0.0)
                            def cbody(kk, _, j=j, off0=off0, expav=expav):
                                off = off0 + kk * 16
                                accbuf[pl.ds(off, 16)] = (
                                    accbuf[pl.ds(off, 16)]
                                    + expav * rowsL[j, pl.ds(off, 16)])
                                return 0
                            lax.fori_loop(0, C // 16, cbody, 0)
            return cur

        lax.fori_loop(0, EMAX // 16, gbody, jnp.int32(-1))

    return kern(xl, xr, att_flat, src_bins, dst_bins)


def _pool_body(h_ref, b_ref, G1_ref, g1b_ref, G2_ref, g2b_ref,
               emb_ref, den_ref):
    i = pl.program_id(0)
    hb = h_ref[...]
    g1 = jnp.maximum(jnp.dot(hb, G1_ref[...],
                             preferred_element_type=jnp.float32)
                     + g1b_ref[...], 0.0)
    gate = jnp.sum(g1 * G2_ref[...], axis=1, keepdims=True) + g2b_ref[...]
    eg = jnp.exp(gate)
    bv = b_ref[0, 0, :]
    gi = lax.broadcasted_iota(jnp.int32, (G, hb.shape[0]), 0)
    onehot = (gi == bv[None, :]).astype(jnp.float32)

    @pl.when(i == 0)
    def _():
        emb_ref[...] = jnp.zeros_like(emb_ref)
        den_ref[...] = jnp.zeros_like(den_ref)

    emb_ref[...] += jnp.dot(onehot, eg * hb,
                            preferred_element_type=jnp.float32)
    den_ref[...] += jnp.dot(onehot, jnp.broadcast_to(eg, (hb.shape[0], 128)),
                            preferred_element_type=jnp.float32)


def _pool(h3, batch3, G1, g1b, G2r, g2b, bm=1024):
    nblk = NPAD // bm
    return pl.pallas_call(
        _pool_body,
        grid=(nblk,),
        in_specs=[pl.BlockSpec((bm, 256), lambda i: (i, 0)),
                  pl.BlockSpec((1, 1, bm), lambda i: (i, 0, 0)),
                  pl.BlockSpec((256, 128), lambda i: (0, 0)),
                  pl.BlockSpec((1, 128), lambda i: (0, 0)),
                  pl.BlockSpec((1, 128), lambda i: (0, 0)),
                  pl.BlockSpec((1, 1), lambda i: (0, 0))],
        out_specs=[pl.BlockSpec((G, 256), lambda i: (0, 0)),
                   pl.BlockSpec((G, 128), lambda i: (0, 0))],
        out_shape=[jax.ShapeDtypeStruct((G, 256), jnp.float32),
                   jax.ShapeDtypeStruct((G, 128), jnp.float32)],
    )(h3, batch3, G1, g1b, G2r, g2b)


def _ln(x, w, b, eps=1e-5):
    m = jnp.mean(x, axis=-1, keepdims=True)
    v = jnp.mean((x - m) ** 2, axis=-1, keepdims=True)
    return (x - m) / jnp.sqrt(v + eps) * w + b


def _head_body(emb_ref, den_ref, M1, ln1w, ln1b, M2, ln2w, ln2b,
               M3, ln3w, ln3b, M4, ln4w, ln4b, M5, M5b, o_ref):
    z = emb_ref[...] / (den_ref[...][:, 0:1] + 1e-16)
    z = jnp.maximum(_ln(jnp.dot(z, M1[...]), ln1w[...], ln1b[...]), 0.0)
    z = jnp.maximum(_ln(jnp.dot(z, M2[...]), ln2w[...], ln2b[...]), 0.0)
    z = jnp.maximum(_ln(jnp.dot(z, M3[...]), ln3w[...], ln3b[...]), 0.0)
    z = jnp.maximum(_ln(jnp.dot(z, M4[...]), ln4w[...], ln4b[...]), 0.0)
    o_ref[...] = jnp.dot(z, M5[...]) + M5b[...]


def _head(emb, den, M1, ln1w, ln1b, M2, ln2w, ln2b, M3, ln3w, ln3b,
          M4, ln4w, ln4b, M5, M5b):
    return pl.pallas_call(
        _head_body,
        out_shape=jax.ShapeDtypeStruct((G, 1), jnp.float32),
    )(emb, den, M1, ln1w, ln1b, M2, ln2w, ln2b, M3, ln3w, ln3b,
      M4, ln4w, ln4b, M5, M5b)


def kernel(x, edge_index, batch, Wl1, Wr1, att1, Wl2, Wr2, att2,
           Wl3, Wr3, att3, G1, g1b, G2, g2b, M1, ln1w, ln1b, M2, ln2w, ln2b,
           M3, ln3w, ln3b, M4, ln4w, ln4b, M5, M5b):
    src = edge_index[0]
    dst = edge_index[1]

    order = jnp.argsort(dst)
    src_s = src[order]
    dst_s = dst[order]
    owner = dst_s // NPW
    starts = jnp.searchsorted(dst_s, jnp.arange(NW, dtype=jnp.int32) * NPW)
    pos = jnp.arange(E, dtype=jnp.int32) - starts[owner].astype(jnp.int32)
    src_bins = jnp.zeros((NW, EMAX), jnp.int32).at[owner, pos].set(src_s)
    dst_bins = jnp.full((NW, EMAX), NPAD - 1, jnp.int32).at[owner, pos].set(dst_s)

    x_pad = jnp.pad(x, ((0, NPAD - N), (0, 0)))

    h = x_pad
    for Wl, Wr, att, C in ((Wl1, Wr1, att1, 1024),
                           (Wl2, Wr2, att2, 512),
                           (Wl3, Wr3, att3, 256)):
        xl = _matmul(h, Wl)
        xr = _matmul(h, Wr)
        h = _edge_conv(xl, xr, att.reshape(-1), src_bins, dst_bins, C)

    batch_pad = jnp.pad(batch, (0, NPAD - N), constant_values=G)
    batch3 = batch_pad.reshape(NPAD // 1024, 1, 1024)
    emb, den = _pool(h, batch3, G1, g1b.reshape(1, 128),
                     jnp.broadcast_to(G2.reshape(1, 128), (1, 128)),
                     g2b.reshape(1, 1))
    return _head(emb, den, M1, ln1w.reshape(1, -1), ln1b.reshape(1, -1),
                 M2, ln2w.reshape(1, -1), ln2b.reshape(1, -1),
                 M3, ln3w.reshape(1, -1), ln3b.reshape(1, -1),
                 M4, ln4w.reshape(1, -1), ln4b.reshape(1, -1),
                 M5, M5b.reshape(1, 1))
